# baseline (speedup 1.0000x reference)
import functools
import jax
import jax.numpy as jnp
from jax import lax
from jax.experimental import pallas as pl
from jax.experimental.pallas import tpu as pltpu
from jax.experimental.pallas import tpu_sc as plsc

N = 10000
NP = 10240
NROWS = NP // 128
E = 640000
NW = 32
CHUNK = 128
KCH = 160
EP = NW * KCH * CHUNK
C2 = 64
SBC = 40
SB0 = 16
SB1 = 0
TOT_CH = 16 * (SB0 + SB1) * SBC
NB = 4

_mesh = plsc.VectorSubcoreMesh(core_axis_name="c", subcore_axis_name="s")


def _deg_body(dst_hbm, out_hbm, dst16, ones_v, hist_sh):
    c = lax.axis_index("c")
    s = lax.axis_index("s")
    w = s * 2 + c

    zero16 = jnp.zeros((16,), jnp.float32)
    ones16 = jnp.ones((16,), jnp.float32)

    def zfill(t, carry):
        ones_v[t // 8, pl.ds((t % 8) * 16, 16)] = zero16
        return carry
    lax.fori_loop(0, CHUNK * 8, zfill, 0)
    for b in range(5):
        pltpu.sync_copy(ones_v, hist_sh.at[pl.ds(s * 640 + b * 128, 128)])

    def ofill(t, carry):
        ones_v[t // 8, pl.ds((t % 8) * 16, 16)] = ones16
        return carry
    lax.fori_loop(0, CHUNK * 8, ofill, 0)
    plsc.subcore_barrier()

    def superblk(t, carry):
        pltpu.sync_copy(dst_hbm.at[w, pl.ds(t * 16, 16)], dst16)

        def chunk(k, carry2):
            pltpu.sync_copy(ones_v, hist_sh.at[dst16.at[k]], add=True)
            return carry2
        lax.fori_loop(0, 16, chunk, 0)
        return carry
    lax.fori_loop(0, KCH // 16, superblk, 0)

    plsc.subcore_barrier()
    for b in range(5):
        pltpu.sync_copy(hist_sh.at[pl.ds(s * 640 + b * 128, 128)], ones_v)
        pltpu.sync_copy(ones_v, out_hbm.at[c, pl.ds(s * 640 + b * 128, 128)])


@functools.partial(
    pl.kernel,
    mesh=_mesh,
    out_type=jax.ShapeDtypeStruct((2, NP, 128), jnp.float32),
    scratch_types=[
        pltpu.VMEM((16, CHUNK), jnp.int32),
        pltpu.VMEM((CHUNK, 128), jnp.float32),
        pltpu.VMEM_SHARED((NP, 128), jnp.float32),
    ],
)
def _deg_kernel(dst_hbm, out_hbm, dst16, ones_v, hist_sh):
    _deg_body(dst_hbm, out_hbm, dst16, ones_v, hist_sh)


def _agg_body(h_hbm, src_hbm, dst_hbm, out_hbm, src_v, dst_v, b0, b1, b2, b3,
              s0, s1, s2, s3, acc_sh):
    c = lax.axis_index("c")
    s = lax.axis_index("s")
    w = s * 2 + c
    bufs = (b0, b1, b2, b3)
    sems = (s0, s1, s2, s3)

    zero16 = jnp.zeros((16,), jnp.float32)

    def zrow(t, carry):
        b0[t // 8, pl.ds((t % 8) * 16, 16)] = zero16
        return carry
    lax.fori_loop(0, C2 * 8, zrow, 0)
    for b in range(10):
        pltpu.sync_copy(b0, acc_sh.at[pl.ds(s * 640 + b * 64, 64)])
    plsc.subcore_barrier()

    nsb = SB0 + c * (SB1 - SB0)
    cbase = s * SBC * (SB0 + c * (SB1 - SB0)) + c * 16 * SB0 * SBC

    def superblk(t, carry):
        off = pl.multiple_of(cbase + t * SBC, 8)
        pltpu.sync_copy(src_hbm.at[pl.ds(off, SBC)], src_v)
        pltpu.sync_copy(dst_hbm.at[pl.ds(off, SBC)], dst_v)
        for u in range(NB - 1):
            pltpu.async_copy(h_hbm.at[src_v.at[u]], bufs[u], sems[u])

        def group(g, carry2):
            for u in range(NB):
                k = NB * g + u
                pltpu.make_async_copy(
                    h_hbm.at[src_v.at[k]], bufs[u], sems[u]).wait()

                un = (u + NB - 1) % NB

                @pl.when(k + NB - 1 < SBC)
                def _start(k=k, un=un):
                    pltpu.async_copy(
                        h_hbm.at[src_v.at[k + NB - 1]], bufs[un], sems[un])

                pltpu.sync_copy(bufs[u], acc_sh.at[dst_v.at[k]], add=True)
            return carry2
        lax.fori_loop(0, SBC // NB, group, 0)
        return carry
    lax.fori_loop(0, nsb, superblk, 0)

    plsc.subcore_barrier()
    for b in range(10):
        pltpu.sync_copy(acc_sh.at[pl.ds(s * 640 + b * 64, 64)], b0)
        pltpu.sync_copy(b0, out_hbm.at[c, pl.ds(s * 640 + b * 64, 64)])


@functools.partial(
    pl.kernel,
    mesh=_mesh,
    out_type=jax.ShapeDtypeStruct((2, NP, 128), jnp.float32),
    scratch_types=[
        pltpu.VMEM((SBC, C2), jnp.int32),
        pltpu.VMEM((SBC, C2), jnp.int32),
        pltpu.VMEM((C2, 128), jnp.float32),
        pltpu.VMEM((C2, 128), jnp.float32),
        pltpu.VMEM((C2, 128), jnp.float32),
        pltpu.VMEM((C2, 128), jnp.float32),
        pltpu.SemaphoreType.DMA,
        pltpu.SemaphoreType.DMA,
        pltpu.SemaphoreType.DMA,
        pltpu.SemaphoreType.DMA,
        pltpu.VMEM_SHARED((NP, 128), jnp.float32),
    ],
)
def _agg_kernel(h_hbm, src_hbm, dst_hbm, out_hbm, src_v, dst_v, b0, b1, b2, b3,
                s0, s1, s2, s3, acc_sh):
    _agg_body(h_hbm, src_hbm, dst_hbm, out_hbm, src_v, dst_v, b0, b1, b2, b3,
              s0, s1, s2, s3, acc_sh)


_RB = 512
_GRID = NP // _RB


def _lin1_tc(x_ref, w_ref, d0_ref, d1_ref, h_ref, dinv_ref):
    dinv = lax.rsqrt(d0_ref[...] + d1_ref[...] + 1.0)
    h = jnp.dot(x_ref[...], w_ref[...], preferred_element_type=jnp.float32)
    h_ref[...] = h * dinv
    dinv_ref[...] = dinv


def _mid_tc(a0_ref, a1_ref, hs_ref, dinv_ref, b_ref, w_ref, out_ref):
    dv = dinv_ref[...]
    o1 = (a0_ref[...] + a1_ref[...] + hs_ref[...]) * dv + b_ref[...]
    out_ref[...] = jnp.dot(o1, w_ref[...], preferred_element_type=jnp.float32) * dv


def _fin_tc(a0_ref, a1_ref, hs_ref, dinv_ref, b_ref, w_ref, bl_ref, out_ref):
    dv = dinv_ref[...]
    o1 = (a0_ref[...] + a1_ref[...] + hs_ref[...]) * dv + b_ref[...]
    out_ref[...] = (
        jnp.dot(o1, w_ref[...], preferred_element_type=jnp.float32) + bl_ref[...]
    )


def _rows_spec():
    return pl.BlockSpec((_RB, 128), lambda i: (i, 0))


def _col_spec():
    return pl.BlockSpec((_RB, 1), lambda i: (i, 0))


def _full_spec():
    return pl.BlockSpec((128, 128), lambda i: (0, 0))


def _bias_spec():
    return pl.BlockSpec((1, 128), lambda i: (0, 0))


_tc_lin1 = pl.pallas_call(
    _lin1_tc,
    grid=(_GRID,),
    in_specs=[_rows_spec(), _full_spec(), _col_spec(), _col_spec()],
    out_specs=[_rows_spec(), _col_spec()],
    out_shape=[
        jax.ShapeDtypeStruct((NP, 128), jnp.float32),
        jax.ShapeDtypeStruct((NP, 1), jnp.float32),
    ],
)

_tc_mid = pl.pallas_call(
    _mid_tc,
    grid=(_GRID,),
    in_specs=[_rows_spec(), _rows_spec(), _rows_spec(), _col_spec(),
              _bias_spec(), _full_spec()],
    out_specs=_rows_spec(),
    out_shape=jax.ShapeDtypeStruct((NP, 128), jnp.float32),
)

_tc_fin = pl.pallas_call(
    _fin_tc,
    grid=(_GRID,),
    in_specs=[_rows_spec(), _rows_spec(), _rows_spec(), _col_spec(),
              _bias_spec(), _full_spec(), _bias_spec()],
    out_specs=_rows_spec(),
    out_shape=jax.ShapeDtypeStruct((NP, 128), jnp.float32),
)


def kernel(x, edge_index, W1, b1, W2, b2, Wl, bl):
    xp = jnp.pad(x, ((0, NP - N), (0, 0)))
    pad = jnp.full((EP - E,), N, jnp.int32)
    src = jnp.concatenate([edge_index[0], pad])
    dst = jnp.concatenate([edge_index[1], pad])
    srcp = src.reshape(TOT_CH, C2)
    dstp = dst.reshape(TOT_CH, C2)
    dstp128 = dst.reshape(NW, KCH, CHUNK)

    degp = _deg_kernel(dstp128)
    d0 = degp[0, :, 0:1]
    d1 = degp[1, :, 0:1]

    h1s, dinv = _tc_lin1(xp, W1, d0, d1)
    a1 = _agg_kernel(h1s, srcp, dstp)
    h2s = _tc_mid(a1[0], a1[1], h1s, dinv, b1.reshape(1, 128), W2)
    a2 = _agg_kernel(h2s, srcp, dstp)
    out = _tc_fin(a2[0], a2[1], h2s, dinv, b2.reshape(1, 128), Wl,
                  bl.reshape(1, 128))
    return out[:N]


# speedup vs baseline: 1.2389x; 1.2389x over previous
import functools
import jax
import jax.numpy as jnp
from jax import lax
from jax.experimental import pallas as pl
from jax.experimental.pallas import tpu as pltpu
from jax.experimental.pallas import tpu_sc as plsc

N = 10000
NP = 10240
NROWS = NP // 128
E = 640000
NW = 32
CHUNK = 128
KCH = 160
EP = NW * KCH * CHUNK
C2 = 64
SBC = 40
SB0 = 12
SB1 = 4
TOT_CH = 16 * (SB0 + SB1) * SBC
NB = 4

_mesh = plsc.VectorSubcoreMesh(core_axis_name="c", subcore_axis_name="s")


def _deg_body(dst_hbm, out_hbm, dst16, ones_v, hist_sh):
    c = lax.axis_index("c")
    s = lax.axis_index("s")
    w = s * 2 + c

    zero16 = jnp.zeros((16,), jnp.float32)
    ones16 = jnp.ones((16,), jnp.float32)

    def zfill(t, carry):
        ones_v[t // 8, pl.ds((t % 8) * 16, 16)] = zero16
        return carry
    lax.fori_loop(0, CHUNK * 8, zfill, 0)
    for b in range(5):
        pltpu.sync_copy(ones_v, hist_sh.at[pl.ds(s * 640 + b * 128, 128)])

    def ofill(t, carry):
        ones_v[t // 8, pl.ds((t % 8) * 16, 16)] = ones16
        return carry
    lax.fori_loop(0, CHUNK * 8, ofill, 0)
    plsc.subcore_barrier()

    def superblk(t, carry):
        pltpu.sync_copy(dst_hbm.at[w, pl.ds(t * 16, 16)], dst16)

        def chunk(k, carry2):
            pltpu.sync_copy(ones_v, hist_sh.at[dst16.at[k]], add=True)
            return carry2
        lax.fori_loop(0, 16, chunk, 0)
        return carry
    lax.fori_loop(0, KCH // 16, superblk, 0)

    plsc.subcore_barrier()
    for b in range(5):
        pltpu.sync_copy(hist_sh.at[pl.ds(s * 640 + b * 128, 128)], ones_v)
        pltpu.sync_copy(ones_v, out_hbm.at[c, pl.ds(s * 640 + b * 128, 128)])


@functools.partial(
    pl.kernel,
    mesh=_mesh,
    out_type=jax.ShapeDtypeStruct((2, NP, 128), jnp.float32),
    scratch_types=[
        pltpu.VMEM((16, CHUNK), jnp.int32),
        pltpu.VMEM((CHUNK, 128), jnp.float32),
        pltpu.VMEM_SHARED((NP, 128), jnp.float32),
    ],
)
def _deg_kernel(dst_hbm, out_hbm, dst16, ones_v, hist_sh):
    _deg_body(dst_hbm, out_hbm, dst16, ones_v, hist_sh)


def _agg_body(h_hbm, src_hbm, dst_hbm, out_hbm, src_v, dst_v, b0, b1, b2, b3,
              s0, s1, s2, s3, acc_sh):
    c = lax.axis_index("c")
    s = lax.axis_index("s")
    w = s * 2 + c
    bufs = (b0, b1, b2, b3)
    sems = (s0, s1, s2, s3)

    zero16 = jnp.zeros((16,), jnp.float32)

    def zrow(t, carry):
        b0[t // 8, pl.ds((t % 8) * 16, 16)] = zero16
        return carry
    lax.fori_loop(0, C2 * 8, zrow, 0)
    for b in range(10):
        pltpu.sync_copy(b0, acc_sh.at[pl.ds(s * 640 + b * 64, 64)])
    plsc.subcore_barrier()

    nsb = SB0 + c * (SB1 - SB0)
    cbase = s * SBC * (SB0 + c * (SB1 - SB0)) + c * 16 * SB0 * SBC

    def superblk(t, carry):
        off = pl.multiple_of(cbase + t * SBC, 8)
        pltpu.sync_copy(src_hbm.at[pl.ds(off, SBC)], src_v)
        pltpu.sync_copy(dst_hbm.at[pl.ds(off, SBC)], dst_v)
        for u in range(NB - 1):
            pltpu.async_copy(h_hbm.at[src_v.at[u]], bufs[u], sems[u])

        def group(g, carry2):
            for u in range(NB):
                k = NB * g + u
                pltpu.make_async_copy(
                    h_hbm.at[src_v.at[k]], bufs[u], sems[u]).wait()

                un = (u + NB - 1) % NB

                @pl.when(k + NB - 1 < SBC)
                def _start(k=k, un=un):
                    pltpu.async_copy(
                        h_hbm.at[src_v.at[k + NB - 1]], bufs[un], sems[un])

                pltpu.sync_copy(bufs[u], acc_sh.at[dst_v.at[k]], add=True)
            return carry2
        lax.fori_loop(0, SBC // NB, group, 0)
        return carry
    lax.fori_loop(0, nsb, superblk, 0)

    plsc.subcore_barrier()
    for b in range(10):
        pltpu.sync_copy(acc_sh.at[pl.ds(s * 640 + b * 64, 64)], b0)
        pltpu.sync_copy(b0, out_hbm.at[c, pl.ds(s * 640 + b * 64, 64)])


@functools.partial(
    pl.kernel,
    mesh=_mesh,
    out_type=jax.ShapeDtypeStruct((2, NP, 128), jnp.float32),
    scratch_types=[
        pltpu.VMEM((SBC, C2), jnp.int32),
        pltpu.VMEM((SBC, C2), jnp.int32),
        pltpu.VMEM((C2, 128), jnp.float32),
        pltpu.VMEM((C2, 128), jnp.float32),
        pltpu.VMEM((C2, 128), jnp.float32),
        pltpu.VMEM((C2, 128), jnp.float32),
        pltpu.SemaphoreType.DMA,
        pltpu.SemaphoreType.DMA,
        pltpu.SemaphoreType.DMA,
        pltpu.SemaphoreType.DMA,
        pltpu.VMEM_SHARED((NP, 128), jnp.float32),
    ],
)
def _agg_kernel(h_hbm, src_hbm, dst_hbm, out_hbm, src_v, dst_v, b0, b1, b2, b3,
                s0, s1, s2, s3, acc_sh):
    _agg_body(h_hbm, src_hbm, dst_hbm, out_hbm, src_v, dst_v, b0, b1, b2, b3,
              s0, s1, s2, s3, acc_sh)


_RB = 512
_GRID = NP // _RB


def _lin1_tc(x_ref, w_ref, d0_ref, d1_ref, h_ref, dinv_ref):
    dinv = lax.rsqrt(d0_ref[...] + d1_ref[...] + 1.0)
    h = jnp.dot(x_ref[...], w_ref[...], preferred_element_type=jnp.float32)
    h_ref[...] = h * dinv
    dinv_ref[...] = dinv


def _mid_tc(a0_ref, a1_ref, hs_ref, dinv_ref, b_ref, w_ref, out_ref):
    dv = dinv_ref[...]
    o1 = (a0_ref[...] + a1_ref[...] + hs_ref[...]) * dv + b_ref[...]
    out_ref[...] = jnp.dot(o1, w_ref[...], preferred_element_type=jnp.float32) * dv


def _fin_tc(a0_ref, a1_ref, hs_ref, dinv_ref, b_ref, w_ref, bl_ref, out_ref):
    dv = dinv_ref[...]
    o1 = (a0_ref[...] + a1_ref[...] + hs_ref[...]) * dv + b_ref[...]
    out_ref[...] = (
        jnp.dot(o1, w_ref[...], preferred_element_type=jnp.float32) + bl_ref[...]
    )


def _rows_spec():
    return pl.BlockSpec((_RB, 128), lambda i: (i, 0))


def _col_spec():
    return pl.BlockSpec((_RB, 1), lambda i: (i, 0))


def _full_spec():
    return pl.BlockSpec((128, 128), lambda i: (0, 0))


def _bias_spec():
    return pl.BlockSpec((1, 128), lambda i: (0, 0))


_tc_lin1 = pl.pallas_call(
    _lin1_tc,
    grid=(_GRID,),
    in_specs=[_rows_spec(), _full_spec(), _col_spec(), _col_spec()],
    out_specs=[_rows_spec(), _col_spec()],
    out_shape=[
        jax.ShapeDtypeStruct((NP, 128), jnp.float32),
        jax.ShapeDtypeStruct((NP, 1), jnp.float32),
    ],
)

_tc_mid = pl.pallas_call(
    _mid_tc,
    grid=(_GRID,),
    in_specs=[_rows_spec(), _rows_spec(), _rows_spec(), _col_spec(),
              _bias_spec(), _full_spec()],
    out_specs=_rows_spec(),
    out_shape=jax.ShapeDtypeStruct((NP, 128), jnp.float32),
)

_tc_fin = pl.pallas_call(
    _fin_tc,
    grid=(_GRID,),
    in_specs=[_rows_spec(), _rows_spec(), _rows_spec(), _col_spec(),
              _bias_spec(), _full_spec(), _bias_spec()],
    out_specs=_rows_spec(),
    out_shape=jax.ShapeDtypeStruct((NP, 128), jnp.float32),
)


def kernel(x, edge_index, W1, b1, W2, b2, Wl, bl):
    xp = jnp.pad(x, ((0, NP - N), (0, 0)))
    pad = jnp.full((EP - E,), N, jnp.int32)
    src = jnp.concatenate([edge_index[0], pad])
    dst = jnp.concatenate([edge_index[1], pad])
    srcp = src.reshape(TOT_CH, C2)
    dstp = dst.reshape(TOT_CH, C2)
    dstp128 = dst.reshape(NW, KCH, CHUNK)

    degp = _deg_kernel(dstp128)
    d0 = degp[0, :, 0:1]
    d1 = degp[1, :, 0:1]

    h1s, dinv = _tc_lin1(xp, W1, d0, d1)
    a1 = _agg_kernel(h1s, srcp, dstp)
    h2s = _tc_mid(a1[0], a1[1], h1s, dinv, b1.reshape(1, 128), W2)
    a2 = _agg_kernel(h2s, srcp, dstp)
    out = _tc_fin(a2[0], a2[1], h2s, dinv, b2.reshape(1, 128), Wl,
                  bl.reshape(1, 128))
    return out[:N]


# speedup vs baseline: 1.2740x; 1.0283x over previous
import functools
import jax
import jax.numpy as jnp
from jax import lax
from jax.experimental import pallas as pl
from jax.experimental.pallas import tpu as pltpu
from jax.experimental.pallas import tpu_sc as plsc

N = 10000
NP = 10240
NROWS = NP // 128
E = 640000
NW = 32
CHUNK = 128
KCH = 160
EP = NW * KCH * CHUNK
C2 = 64
SBC = 40
SB0 = 14
SB1 = 2
TOT_CH = 16 * (SB0 + SB1) * SBC
NB = 4

_mesh = plsc.VectorSubcoreMesh(core_axis_name="c", subcore_axis_name="s")


def _deg_body(dst_hbm, out_hbm, dst16, ones_v, hist_sh):
    c = lax.axis_index("c")
    s = lax.axis_index("s")
    w = s * 2 + c

    zero16 = jnp.zeros((16,), jnp.float32)
    ones16 = jnp.ones((16,), jnp.float32)

    def zfill(t, carry):
        ones_v[t // 8, pl.ds((t % 8) * 16, 16)] = zero16
        return carry
    lax.fori_loop(0, CHUNK * 8, zfill, 0)
    for b in range(5):
        pltpu.sync_copy(ones_v, hist_sh.at[pl.ds(s * 640 + b * 128, 128)])

    def ofill(t, carry):
        ones_v[t // 8, pl.ds((t % 8) * 16, 16)] = ones16
        return carry
    lax.fori_loop(0, CHUNK * 8, ofill, 0)
    plsc.subcore_barrier()

    def superblk(t, carry):
        pltpu.sync_copy(dst_hbm.at[w, pl.ds(t * 16, 16)], dst16)

        def chunk(k, carry2):
            pltpu.sync_copy(ones_v, hist_sh.at[dst16.at[k]], add=True)
            return carry2
        lax.fori_loop(0, 16, chunk, 0)
        return carry
    lax.fori_loop(0, KCH // 16, superblk, 0)

    plsc.subcore_barrier()
    for b in range(5):
        pltpu.sync_copy(hist_sh.at[pl.ds(s * 640 + b * 128, 128)], ones_v)
        pltpu.sync_copy(ones_v, out_hbm.at[c, pl.ds(s * 640 + b * 128, 128)])


@functools.partial(
    pl.kernel,
    mesh=_mesh,
    out_type=jax.ShapeDtypeStruct((2, NP, 128), jnp.float32),
    scratch_types=[
        pltpu.VMEM((16, CHUNK), jnp.int32),
        pltpu.VMEM((CHUNK, 128), jnp.float32),
        pltpu.VMEM_SHARED((NP, 128), jnp.float32),
    ],
)
def _deg_kernel(dst_hbm, out_hbm, dst16, ones_v, hist_sh):
    _deg_body(dst_hbm, out_hbm, dst16, ones_v, hist_sh)


def _agg_body(h_hbm, src_hbm, dst_hbm, out_hbm, src_v, dst_v, b0, b1, b2, b3,
              s0, s1, s2, s3, acc_sh):
    c = lax.axis_index("c")
    s = lax.axis_index("s")
    w = s * 2 + c
    bufs = (b0, b1, b2, b3)
    sems = (s0, s1, s2, s3)

    zero16 = jnp.zeros((16,), jnp.float32)

    def zrow(t, carry):
        b0[t // 8, pl.ds((t % 8) * 16, 16)] = zero16
        return carry
    lax.fori_loop(0, C2 * 8, zrow, 0)
    for b in range(10):
        pltpu.sync_copy(b0, acc_sh.at[pl.ds(s * 640 + b * 64, 64)])
    plsc.subcore_barrier()

    nsb = SB0 + c * (SB1 - SB0)
    cbase = s * SBC * (SB0 + c * (SB1 - SB0)) + c * 16 * SB0 * SBC

    def superblk(t, carry):
        off = pl.multiple_of(cbase + t * SBC, 8)
        pltpu.sync_copy(src_hbm.at[pl.ds(off, SBC)], src_v)
        pltpu.sync_copy(dst_hbm.at[pl.ds(off, SBC)], dst_v)
        for u in range(NB - 1):
            pltpu.async_copy(h_hbm.at[src_v.at[u]], bufs[u], sems[u])

        def group(g, carry2):
            for u in range(NB):
                k = NB * g + u
                pltpu.make_async_copy(
                    h_hbm.at[src_v.at[k]], bufs[u], sems[u]).wait()

                un = (u + NB - 1) % NB

                @pl.when(k + NB - 1 < SBC)
                def _start(k=k, un=un):
                    pltpu.async_copy(
                        h_hbm.at[src_v.at[k + NB - 1]], bufs[un], sems[un])

                pltpu.sync_copy(bufs[u], acc_sh.at[dst_v.at[k]], add=True)
            return carry2
        lax.fori_loop(0, SBC // NB, group, 0)
        return carry
    lax.fori_loop(0, nsb, superblk, 0)

    plsc.subcore_barrier()
    for b in range(10):
        pltpu.sync_copy(acc_sh.at[pl.ds(s * 640 + b * 64, 64)], b0)
        pltpu.sync_copy(b0, out_hbm.at[c, pl.ds(s * 640 + b * 64, 64)])


@functools.partial(
    pl.kernel,
    mesh=_mesh,
    out_type=jax.ShapeDtypeStruct((2, NP, 128), jnp.float32),
    scratch_types=[
        pltpu.VMEM((SBC, C2), jnp.int32),
        pltpu.VMEM((SBC, C2), jnp.int32),
        pltpu.VMEM((C2, 128), jnp.float32),
        pltpu.VMEM((C2, 128), jnp.float32),
        pltpu.VMEM((C2, 128), jnp.float32),
        pltpu.VMEM((C2, 128), jnp.float32),
        pltpu.SemaphoreType.DMA,
        pltpu.SemaphoreType.DMA,
        pltpu.SemaphoreType.DMA,
        pltpu.SemaphoreType.DMA,
        pltpu.VMEM_SHARED((NP, 128), jnp.float32),
    ],
)
def _agg_kernel(h_hbm, src_hbm, dst_hbm, out_hbm, src_v, dst_v, b0, b1, b2, b3,
                s0, s1, s2, s3, acc_sh):
    _agg_body(h_hbm, src_hbm, dst_hbm, out_hbm, src_v, dst_v, b0, b1, b2, b3,
              s0, s1, s2, s3, acc_sh)


_RB = 512
_GRID = NP // _RB


def _lin1_tc(x_ref, w_ref, d0_ref, d1_ref, h_ref, dinv_ref):
    dinv = lax.rsqrt(d0_ref[...] + d1_ref[...] + 1.0)
    h = jnp.dot(x_ref[...], w_ref[...], preferred_element_type=jnp.float32)
    h_ref[...] = h * dinv
    dinv_ref[...] = dinv


def _mid_tc(a0_ref, a1_ref, hs_ref, dinv_ref, b_ref, w_ref, out_ref):
    dv = dinv_ref[...]
    o1 = (a0_ref[...] + a1_ref[...] + hs_ref[...]) * dv + b_ref[...]
    out_ref[...] = jnp.dot(o1, w_ref[...], preferred_element_type=jnp.float32) * dv


def _fin_tc(a0_ref, a1_ref, hs_ref, dinv_ref, b_ref, w_ref, bl_ref, out_ref):
    dv = dinv_ref[...]
    o1 = (a0_ref[...] + a1_ref[...] + hs_ref[...]) * dv + b_ref[...]
    out_ref[...] = (
        jnp.dot(o1, w_ref[...], preferred_element_type=jnp.float32) + bl_ref[...]
    )


def _rows_spec():
    return pl.BlockSpec((_RB, 128), lambda i: (i, 0))


def _col_spec():
    return pl.BlockSpec((_RB, 1), lambda i: (i, 0))


def _full_spec():
    return pl.BlockSpec((128, 128), lambda i: (0, 0))


def _bias_spec():
    return pl.BlockSpec((1, 128), lambda i: (0, 0))


_tc_lin1 = pl.pallas_call(
    _lin1_tc,
    grid=(_GRID,),
    in_specs=[_rows_spec(), _full_spec(), _col_spec(), _col_spec()],
    out_specs=[_rows_spec(), _col_spec()],
    out_shape=[
        jax.ShapeDtypeStruct((NP, 128), jnp.float32),
        jax.ShapeDtypeStruct((NP, 1), jnp.float32),
    ],
)

_tc_mid = pl.pallas_call(
    _mid_tc,
    grid=(_GRID,),
    in_specs=[_rows_spec(), _rows_spec(), _rows_spec(), _col_spec(),
              _bias_spec(), _full_spec()],
    out_specs=_rows_spec(),
    out_shape=jax.ShapeDtypeStruct((NP, 128), jnp.float32),
)

_tc_fin = pl.pallas_call(
    _fin_tc,
    grid=(_GRID,),
    in_specs=[_rows_spec(), _rows_spec(), _rows_spec(), _col_spec(),
              _bias_spec(), _full_spec(), _bias_spec()],
    out_specs=_rows_spec(),
    out_shape=jax.ShapeDtypeStruct((NP, 128), jnp.float32),
)


def kernel(x, edge_index, W1, b1, W2, b2, Wl, bl):
    xp = jnp.pad(x, ((0, NP - N), (0, 0)))
    pad = jnp.full((EP - E,), N, jnp.int32)
    src = jnp.concatenate([edge_index[0], pad])
    dst = jnp.concatenate([edge_index[1], pad])
    srcp = src.reshape(TOT_CH, C2)
    dstp = dst.reshape(TOT_CH, C2)
    dstp128 = dst.reshape(NW, KCH, CHUNK)

    degp = _deg_kernel(dstp128)
    d0 = degp[0, :, 0:1]
    d1 = degp[1, :, 0:1]

    h1s, dinv = _tc_lin1(xp, W1, d0, d1)
    a1 = _agg_kernel(h1s, srcp, dstp)
    h2s = _tc_mid(a1[0], a1[1], h1s, dinv, b1.reshape(1, 128), W2)
    a2 = _agg_kernel(h2s, srcp, dstp)
    out = _tc_fin(a2[0], a2[1], h2s, dinv, b2.reshape(1, 128), Wl,
                  bl.reshape(1, 128))
    return out[:N]


# speedup vs baseline: 1.3058x; 1.0250x over previous
import functools
import jax
import jax.numpy as jnp
from jax import lax
from jax.experimental import pallas as pl
from jax.experimental.pallas import tpu as pltpu
from jax.experimental.pallas import tpu_sc as plsc

N = 10000
NP = 10240
NROWS = NP // 128
E = 640000
NW = 32
CHUNK = 128
KCH = 160
EP = NW * KCH * CHUNK
C2 = 64
SBC = 40
SB0 = 15
SB1 = 1
TOT_CH = 16 * (SB0 + SB1) * SBC
NB = 4

_mesh = plsc.VectorSubcoreMesh(core_axis_name="c", subcore_axis_name="s")


def _deg_body(dst_hbm, out_hbm, dst16, ones_v, hist_sh):
    c = lax.axis_index("c")
    s = lax.axis_index("s")
    w = s * 2 + c

    zero16 = jnp.zeros((16,), jnp.float32)
    ones16 = jnp.ones((16,), jnp.float32)

    def zfill(t, carry):
        ones_v[t // 8, pl.ds((t % 8) * 16, 16)] = zero16
        return carry
    lax.fori_loop(0, CHUNK * 8, zfill, 0)
    for b in range(5):
        pltpu.sync_copy(ones_v, hist_sh.at[pl.ds(s * 640 + b * 128, 128)])

    def ofill(t, carry):
        ones_v[t // 8, pl.ds((t % 8) * 16, 16)] = ones16
        return carry
    lax.fori_loop(0, CHUNK * 8, ofill, 0)
    plsc.subcore_barrier()

    def superblk(t, carry):
        pltpu.sync_copy(dst_hbm.at[w, pl.ds(t * 16, 16)], dst16)

        def chunk(k, carry2):
            pltpu.sync_copy(ones_v, hist_sh.at[dst16.at[k]], add=True)
            return carry2
        lax.fori_loop(0, 16, chunk, 0)
        return carry
    lax.fori_loop(0, KCH // 16, superblk, 0)

    plsc.subcore_barrier()
    for b in range(5):
        pltpu.sync_copy(hist_sh.at[pl.ds(s * 640 + b * 128, 128)], ones_v)
        pltpu.sync_copy(ones_v, out_hbm.at[c, pl.ds(s * 640 + b * 128, 128)])


@functools.partial(
    pl.kernel,
    mesh=_mesh,
    out_type=jax.ShapeDtypeStruct((2, NP, 128), jnp.float32),
    scratch_types=[
        pltpu.VMEM((16, CHUNK), jnp.int32),
        pltpu.VMEM((CHUNK, 128), jnp.float32),
        pltpu.VMEM_SHARED((NP, 128), jnp.float32),
    ],
)
def _deg_kernel(dst_hbm, out_hbm, dst16, ones_v, hist_sh):
    _deg_body(dst_hbm, out_hbm, dst16, ones_v, hist_sh)


def _agg_body(h_hbm, src_hbm, dst_hbm, out_hbm, src_v, dst_v, b0, b1, b2, b3,
              s0, s1, s2, s3, acc_sh):
    c = lax.axis_index("c")
    s = lax.axis_index("s")
    w = s * 2 + c
    bufs = (b0, b1, b2, b3)
    sems = (s0, s1, s2, s3)

    zero16 = jnp.zeros((16,), jnp.float32)

    def zrow(t, carry):
        b0[t // 8, pl.ds((t % 8) * 16, 16)] = zero16
        return carry
    lax.fori_loop(0, C2 * 8, zrow, 0)
    for b in range(10):
        pltpu.sync_copy(b0, acc_sh.at[pl.ds(s * 640 + b * 64, 64)])
    plsc.subcore_barrier()

    nsb = SB0 + c * (SB1 - SB0)
    cbase = s * SBC * (SB0 + c * (SB1 - SB0)) + c * 16 * SB0 * SBC

    def superblk(t, carry):
        off = pl.multiple_of(cbase + t * SBC, 8)
        pltpu.sync_copy(src_hbm.at[pl.ds(off, SBC)], src_v)
        pltpu.sync_copy(dst_hbm.at[pl.ds(off, SBC)], dst_v)
        for u in range(NB - 1):
            pltpu.async_copy(h_hbm.at[src_v.at[u]], bufs[u], sems[u])

        def group(g, carry2):
            for u in range(NB):
                k = NB * g + u
                pltpu.make_async_copy(
                    h_hbm.at[src_v.at[k]], bufs[u], sems[u]).wait()

                un = (u + NB - 1) % NB

                @pl.when(k + NB - 1 < SBC)
                def _start(k=k, un=un):
                    pltpu.async_copy(
                        h_hbm.at[src_v.at[k + NB - 1]], bufs[un], sems[un])

                pltpu.sync_copy(bufs[u], acc_sh.at[dst_v.at[k]], add=True)
            return carry2
        lax.fori_loop(0, SBC // NB, group, 0)
        return carry
    lax.fori_loop(0, nsb, superblk, 0)

    plsc.subcore_barrier()
    for b in range(10):
        pltpu.sync_copy(acc_sh.at[pl.ds(s * 640 + b * 64, 64)], b0)
        pltpu.sync_copy(b0, out_hbm.at[c, pl.ds(s * 640 + b * 64, 64)])


@functools.partial(
    pl.kernel,
    mesh=_mesh,
    out_type=jax.ShapeDtypeStruct((2, NP, 128), jnp.float32),
    scratch_types=[
        pltpu.VMEM((SBC, C2), jnp.int32),
        pltpu.VMEM((SBC, C2), jnp.int32),
        pltpu.VMEM((C2, 128), jnp.float32),
        pltpu.VMEM((C2, 128), jnp.float32),
        pltpu.VMEM((C2, 128), jnp.float32),
        pltpu.VMEM((C2, 128), jnp.float32),
        pltpu.SemaphoreType.DMA,
        pltpu.SemaphoreType.DMA,
        pltpu.SemaphoreType.DMA,
        pltpu.SemaphoreType.DMA,
        pltpu.VMEM_SHARED((NP, 128), jnp.float32),
    ],
)
def _agg_kernel(h_hbm, src_hbm, dst_hbm, out_hbm, src_v, dst_v, b0, b1, b2, b3,
                s0, s1, s2, s3, acc_sh):
    _agg_body(h_hbm, src_hbm, dst_hbm, out_hbm, src_v, dst_v, b0, b1, b2, b3,
              s0, s1, s2, s3, acc_sh)


_RB = 512
_GRID = NP // _RB


def _lin1_tc(x_ref, w_ref, d0_ref, d1_ref, h_ref, dinv_ref):
    dinv = lax.rsqrt(d0_ref[...] + d1_ref[...] + 1.0)
    h = jnp.dot(x_ref[...], w_ref[...], preferred_element_type=jnp.float32)
    h_ref[...] = h * dinv
    dinv_ref[...] = dinv


def _mid_tc(a0_ref, a1_ref, hs_ref, dinv_ref, b_ref, w_ref, out_ref):
    dv = dinv_ref[...]
    o1 = (a0_ref[...] + a1_ref[...] + hs_ref[...]) * dv + b_ref[...]
    out_ref[...] = jnp.dot(o1, w_ref[...], preferred_element_type=jnp.float32) * dv


def _fin_tc(a0_ref, a1_ref, hs_ref, dinv_ref, b_ref, w_ref, bl_ref, out_ref):
    dv = dinv_ref[...]
    o1 = (a0_ref[...] + a1_ref[...] + hs_ref[...]) * dv + b_ref[...]
    out_ref[...] = (
        jnp.dot(o1, w_ref[...], preferred_element_type=jnp.float32) + bl_ref[...]
    )


def _rows_spec():
    return pl.BlockSpec((_RB, 128), lambda i: (i, 0))


def _col_spec():
    return pl.BlockSpec((_RB, 1), lambda i: (i, 0))


def _full_spec():
    return pl.BlockSpec((128, 128), lambda i: (0, 0))


def _bias_spec():
    return pl.BlockSpec((1, 128), lambda i: (0, 0))


_tc_lin1 = pl.pallas_call(
    _lin1_tc,
    grid=(_GRID,),
    in_specs=[_rows_spec(), _full_spec(), _col_spec(), _col_spec()],
    out_specs=[_rows_spec(), _col_spec()],
    out_shape=[
        jax.ShapeDtypeStruct((NP, 128), jnp.float32),
        jax.ShapeDtypeStruct((NP, 1), jnp.float32),
    ],
)

_tc_mid = pl.pallas_call(
    _mid_tc,
    grid=(_GRID,),
    in_specs=[_rows_spec(), _rows_spec(), _rows_spec(), _col_spec(),
              _bias_spec(), _full_spec()],
    out_specs=_rows_spec(),
    out_shape=jax.ShapeDtypeStruct((NP, 128), jnp.float32),
)

_tc_fin = pl.pallas_call(
    _fin_tc,
    grid=(_GRID,),
    in_specs=[_rows_spec(), _rows_spec(), _rows_spec(), _col_spec(),
              _bias_spec(), _full_spec(), _bias_spec()],
    out_specs=_rows_spec(),
    out_shape=jax.ShapeDtypeStruct((NP, 128), jnp.float32),
)


def kernel(x, edge_index, W1, b1, W2, b2, Wl, bl):
    xp = jnp.pad(x, ((0, NP - N), (0, 0)))
    pad = jnp.full((EP - E,), N, jnp.int32)
    src = jnp.concatenate([edge_index[0], pad])
    dst = jnp.concatenate([edge_index[1], pad])
    srcp = src.reshape(TOT_CH, C2)
    dstp = dst.reshape(TOT_CH, C2)
    dstp128 = dst.reshape(NW, KCH, CHUNK)

    degp = _deg_kernel(dstp128)
    d0 = degp[0, :, 0:1]
    d1 = degp[1, :, 0:1]

    h1s, dinv = _tc_lin1(xp, W1, d0, d1)
    a1 = _agg_kernel(h1s, srcp, dstp)
    h2s = _tc_mid(a1[0], a1[1], h1s, dinv, b1.reshape(1, 128), W2)
    a2 = _agg_kernel(h2s, srcp, dstp)
    out = _tc_fin(a2[0], a2[1], h2s, dinv, b2.reshape(1, 128), Wl,
                  bl.reshape(1, 128))
    return out[:N]
